# Initial kernel scaffold; baseline (speedup 1.0000x reference)
import jax
import jax.numpy as jnp
from jax.experimental import pallas as pl


def kernel(X_deep, session_table, promotion_table, age_table, gender_table, purchase_table, W1, b1, W2, b2, W3, b3, Wf, bf):
    raise NotImplementedError("write your pallas kernel here")


# speedup vs baseline: 1.4199x; 1.4199x over previous
import functools

import jax
import jax.numpy as jnp
from jax import lax
from jax.experimental import pallas as pl
from jax.experimental.pallas import tpu as pltpu
from jax.experimental.pallas import tpu_sc as plsc

_BATCH = 16384
_EMB = 16
_LEN_CONT = 8


def _sc_gather(tables, indices):
    n = len(tables)
    info = plsc.get_sparse_core_info()
    nc, ns = info.num_cores, info.num_subcores
    nw = nc * ns
    bpw = _BATCH // nw

    mesh = plsc.VectorSubcoreMesh(core_axis_name="c", subcore_axis_name="s")
    scratch = (
        [pltpu.VMEM((bpw,), jnp.int32) for _ in range(n)]
        + [pltpu.VMEM((bpw, _EMB), jnp.float32) for _ in range(n)]
        + [pltpu.SemaphoreType.DMA]
    )

    @functools.partial(
        pl.kernel,
        mesh=mesh,
        out_type=tuple(
            jax.ShapeDtypeStruct((_BATCH, _EMB), jnp.float32) for _ in range(n)
        ),
        scratch_types=scratch,
        compiler_params=pltpu.CompilerParams(use_tc_tiling_on_sc=False),
    )
    def k(*refs):
        tabs = refs[:n]
        idxs = refs[n : 2 * n]
        outs = refs[2 * n : 3 * n]
        idx_v = refs[3 * n : 4 * n]
        rows_v = refs[4 * n : 5 * n]
        sem = refs[5 * n]
        wid = lax.axis_index("s") * nc + lax.axis_index("c")
        base = wid * bpw
        for j in range(n):
            pltpu.sync_copy(idxs[j].at[pl.ds(base, bpw)], idx_v[j])
        copies = [
            pltpu.async_copy(tabs[j].at[idx_v[j]], rows_v[j], sem)
            for j in range(n)
        ]
        for c in copies:
            c.wait()
        for j in range(n):
            pltpu.sync_copy(rows_v[j], outs[j].at[pl.ds(base, bpw)])

    return k(*tables, *indices)


def _mlp(embs, cont, W1, b1, W2, b2, W3, b3, Wf, bf):
    blk = 2048
    grid = (_BATCH // blk,)

    def body(e0, e1, e2, e3, e4, c, w1, v1, w2, v2, w3, v3, wf, vf, out):
        x = jnp.concatenate(
            [e0[...], e1[...], e2[...], e3[...], e4[...], c[...]], axis=1
        )
        h = jnp.maximum(
            jnp.dot(x, w1[...], preferred_element_type=jnp.float32) + v1[...], 0.0
        )
        h = jnp.maximum(
            jnp.dot(h, w2[...], preferred_element_type=jnp.float32) + v2[...], 0.0
        )
        h = jnp.maximum(
            jnp.dot(h, w3[...], preferred_element_type=jnp.float32) + v3[...], 0.0
        )
        logit = jnp.dot(h, wf[...], preferred_element_type=jnp.float32) + vf[...]
        out[...] = jax.nn.sigmoid(logit)

    eb = pl.BlockSpec((blk, _EMB), lambda i: (i, 0))
    cb = pl.BlockSpec((blk, _LEN_CONT), lambda i: (i, 0))

    def wspec(shape):
        return pl.BlockSpec(shape, lambda i: (0, 0))

    return pl.pallas_call(
        body,
        grid=grid,
        in_specs=[eb] * 5
        + [cb]
        + [
            wspec((88, 64)),
            wspec((1, 64)),
            wspec((64, 32)),
            wspec((1, 32)),
            wspec((32, 16)),
            wspec((1, 16)),
            wspec((16, 1)),
            wspec((1, 1)),
        ],
        out_specs=pl.BlockSpec((blk, 1), lambda i: (i, 0)),
        out_shape=jax.ShapeDtypeStruct((_BATCH, 1), jnp.float32),
    )(*embs, cont, W1, b1, W2, b2, W3, b3, Wf, bf)


def kernel(X_deep, session_table, promotion_table, age_table, gender_table,
           purchase_table, W1, b1, W2, b2, W3, b3, Wf, bf):
    idxs = [X_deep[:, j] for j in range(5)]
    cont = X_deep[:, 5:].astype(jnp.float32)
    embs = _sc_gather(
        (session_table, promotion_table, age_table, gender_table, purchase_table),
        idxs,
    )
    return _mlp(
        embs, cont,
        W1, b1.reshape(1, 64),
        W2, b2.reshape(1, 32),
        W3, b3.reshape(1, 16),
        Wf, bf.reshape(1, 1),
    )
